# baseline (speedup 1.0000x reference)
import functools

import jax
import jax.numpy as jnp
from jax import lax
from jax.experimental import pallas as pl
from jax.experimental.pallas import tpu as pltpu
from jax.experimental.pallas import tpu_sc as plsc

E = 16
K = 2
D = 2048
N = 4 * 4096
R = 1024
NT = N // R
TB = 4096 // R
NW = 32
RS = N // NW
L = 16
G = RS // L


def _tc_router(x_ref, w_ref, lt_ref, w3_ref, i3_ref):
    lt = lax.dot_general(
        w_ref[...], x_ref[...],
        dimension_numbers=(((1,), (1,)), ((), ())),
        preferred_element_type=jnp.float32,
    )
    lt_ref[...] = lt

    ie = lax.broadcasted_iota(jnp.int32, (E, R), 0)
    m1 = jnp.max(lt, axis=0, keepdims=True)
    i1 = jnp.min(jnp.where(lt == m1, ie, E), axis=0, keepdims=True)
    lt2 = jnp.where(ie == i1, -jnp.inf, lt)
    m2 = jnp.max(lt2, axis=0, keepdims=True)
    i2 = jnp.min(jnp.where(lt2 == m2, ie, E), axis=0, keepdims=True)
    e21 = jnp.exp(m2 - m1)
    den = 1.0 + e21
    w3_ref[...] = jnp.transpose(
        jnp.concatenate((1.0 / den, e21 / den), axis=0)
    )[None]
    i3_ref[...] = jnp.transpose(jnp.concatenate((i1, i2), axis=0))[None]


def _tc_call(xf, w):
    return pl.pallas_call(
        _tc_router,
        grid=(NT,),
        in_specs=[
            pl.BlockSpec((R, D), lambda i: (i, 0)),
            pl.BlockSpec((E, D), lambda i: (0, 0)),
        ],
        out_specs=[
            pl.BlockSpec((E, R), lambda i: (0, i)),
            pl.BlockSpec((1, R, K), lambda i: (i // TB, i % TB, 0)),
            pl.BlockSpec((1, R, K), lambda i: (i // TB, i % TB, 0)),
        ],
        out_shape=[
            jax.ShapeDtypeStruct((E, N), jnp.float32),
            jax.ShapeDtypeStruct((4, 4096, K), jnp.float32),
            jax.ShapeDtypeStruct((4, 4096, K), jnp.int32),
        ],
    )(xf, w)


def _sc_usage(lt_hbm, p_out, lt_v, p_v):
    nc = 2
    wid = lax.axis_index("s") * nc + lax.axis_index("c")
    base = wid * RS
    pltpu.sync_copy(lt_hbm.at[:, pl.ds(base, RS)], lt_v)

    zero = jnp.zeros((L,), jnp.float32)
    accs = [zero] * E

    def group(g, accs):
        sl = pl.ds(g * L, L)
        les = [lt_v[e, sl] for e in range(E)]
        m = les[0]
        for e in range(1, E):
            m = jnp.maximum(m, les[e])
        exps = [jnp.exp(le - m) for le in les]
        s = exps[0]
        for e in range(1, E):
            s = s + exps[e]
        inv = 1.0 / s
        return [acc + ex * inv for acc, ex in zip(accs, exps)]

    accs = lax.fori_loop(0, G, group, accs)
    for e in range(E):
        p_v[e, :] = accs[e]
    pltpu.sync_copy(p_v, p_out.at[wid])


@functools.lru_cache(maxsize=1)
def _sc_call():
    return pl.kernel(
        _sc_usage,
        mesh=plsc.VectorSubcoreMesh(core_axis_name="c", subcore_axis_name="s"),
        out_type=jax.ShapeDtypeStruct((NW, E, L), jnp.float32),
        scratch_types=[
            pltpu.VMEM((E, RS), jnp.float32),
            pltpu.VMEM((E, L), jnp.float32),
        ],
    )


def kernel(x, W):
    b, t, d = x.shape
    xf = x.reshape(b * t, d)
    lt, w3, i3 = _tc_call(xf, W)
    partials = _sc_call()(lt)
    usage = jnp.sum(partials, axis=(0, 2)) * (1.0 / N)
    loss = E * jnp.sum(usage * usage)
    return (w3, i3, loss)


# speedup vs baseline: 1.2489x; 1.2489x over previous
import functools

import jax
import jax.numpy as jnp
from jax import lax
from jax.experimental import pallas as pl
from jax.experimental.pallas import tpu as pltpu
from jax.experimental.pallas import tpu_sc as plsc

E = 16
K = 2
D = 2048
N = 4 * 4096
R = 1024
NT = N // R
NW = 32
RS = N // NW
L = 16
G = RS // L


def _tc_router(x_ref, w_ref, lt_ref, loss_ref, acc_ref):
    i = pl.program_id(0)
    lt = lax.dot_general(
        w_ref[...], x_ref[...],
        dimension_numbers=(((1,), (1,)), ((), ())),
        preferred_element_type=jnp.float32,
    )
    lt_ref[...] = lt
    m = jnp.max(lt, axis=0, keepdims=True)
    p = jnp.exp(lt - m)
    s = jnp.sum(p, axis=0, keepdims=True)
    part = jnp.sum(p / s, axis=1, keepdims=True)

    @pl.when(i == 0)
    def _():
        acc_ref[...] = jnp.zeros_like(acc_ref)

    acc_ref[...] += part

    @pl.when(i == pl.num_programs(0) - 1)
    def _():
        usage = acc_ref[...] * (1.0 / N)
        loss_ref[...] = E * jnp.sum(usage * usage, axis=(0, 1), keepdims=True)


def _tc_call(xf, w):
    return pl.pallas_call(
        _tc_router,
        grid=(NT,),
        in_specs=[
            pl.BlockSpec((R, D), lambda i: (i, 0)),
            pl.BlockSpec((E, D), lambda i: (0, 0)),
        ],
        out_specs=[
            pl.BlockSpec((E, R), lambda i: (0, i)),
            pl.BlockSpec((1, 1), lambda i: (0, 0)),
        ],
        out_shape=[
            jax.ShapeDtypeStruct((E, N), jnp.float32),
            jax.ShapeDtypeStruct((1, 1), jnp.float32),
        ],
        scratch_shapes=[pltpu.VMEM((E, 1), jnp.float32)],
    )(xf, w)


def _sc_router(lt_hbm, w_out, i_out, lt_v, w_v, i_v):
    nc = 2
    wid = lax.axis_index("s") * nc + lax.axis_index("c")
    base = wid * RS
    pltpu.sync_copy(lt_hbm.at[:, pl.ds(base, RS)], lt_v)

    def group(g, _):
        sl = pl.ds(g * L, L)

        def expert(e, carry):
            m1, i1, m2, i2 = carry
            le = lt_v[e, sl]
            es = jnp.full((L,), e, jnp.int32)
            gt1 = le > m1
            gt2 = le > m2
            n_m2 = jnp.where(gt1, m1, jnp.where(gt2, le, m2))
            n_i2 = jnp.where(gt1, i1, jnp.where(gt2, es, i2))
            return (
                jnp.where(gt1, le, m1),
                jnp.where(gt1, es, i1),
                n_m2,
                n_i2,
            )

        init = (
            lt_v[0, sl],
            jnp.zeros((L,), jnp.int32),
            jnp.full((L,), -3.0e38, jnp.float32),
            jnp.zeros((L,), jnp.int32),
        )
        m1, i1, m2, i2 = lax.fori_loop(1, E, expert, init)
        e21 = jnp.exp(m2 - m1)
        den = 1.0 + e21
        w_v[0, sl] = 1.0 / den
        w_v[1, sl] = e21 / den
        i_v[0, sl] = i1
        i_v[1, sl] = i2
        return _

    lax.fori_loop(0, G, group, 0)

    pltpu.sync_copy(w_v, w_out.at[:, pl.ds(base, RS)])
    pltpu.sync_copy(i_v, i_out.at[:, pl.ds(base, RS)])


@functools.lru_cache(maxsize=1)
def _sc_call():
    return pl.kernel(
        _sc_router,
        mesh=plsc.VectorSubcoreMesh(core_axis_name="c", subcore_axis_name="s"),
        out_type=[
            jax.ShapeDtypeStruct((K, N), jnp.float32),
            jax.ShapeDtypeStruct((K, N), jnp.int32),
        ],
        scratch_types=[
            pltpu.VMEM((E, RS), jnp.float32),
            pltpu.VMEM((K, RS), jnp.float32),
            pltpu.VMEM((K, RS), jnp.int32),
        ],
    )


def kernel(x, W):
    b, t, d = x.shape
    xf = x.reshape(b * t, d)
    lt, loss = _tc_call(xf, W)
    ws, idx = _sc_call()(lt)
    return (
        ws.T.reshape(b, t, K),
        idx.T.reshape(b, t, K),
        loss[0, 0],
    )
